# baseline (speedup 1.0000x reference)
import functools

import jax
import jax.numpy as jnp
from jax.experimental import pallas as pl
from jax.experimental.pallas import tpu as pltpu

N_CLS = 10000
N_CTX = 16
D = 512
SEQ = 77
SUF = 60
B = 1024


def _body(idx_ref, pre_ref, ctx_ref, suf_ref, out_ref):
    out_ref[0, 0:1, :] = pre_ref[0]
    out_ref[0, 1:1 + N_CTX, :] = ctx_ref[...]
    out_ref[0, 1 + N_CTX:, :] = suf_ref[0]


@jax.jit
def _launch(idx, ctx, token_prefix, token_suffix):
    grid_spec = pltpu.PrefetchScalarGridSpec(
        num_scalar_prefetch=1,
        grid=(B,),
        in_specs=[
            pl.BlockSpec((1, 1, D), lambda b, idx_ref: (idx_ref[b], 0, 0)),
            pl.BlockSpec((N_CTX, D), lambda b, idx_ref: (0, 0)),
            pl.BlockSpec((1, SUF, D), lambda b, idx_ref: (idx_ref[b], 0, 0)),
        ],
        out_specs=pl.BlockSpec((1, SEQ, D), lambda b, idx_ref: (b, 0, 0)),
    )
    return pl.pallas_call(
        _body,
        grid_spec=grid_spec,
        out_shape=jax.ShapeDtypeStruct((B, SEQ, D), jnp.float32),
    )(idx, token_prefix, ctx, token_suffix)


def kernel(idx, ctx, token_prefix, token_suffix):
    return _launch(idx, ctx, token_prefix, token_suffix)


# speedup vs baseline: 1.3868x; 1.3868x over previous
import jax
import jax.numpy as jnp
from jax.experimental import pallas as pl
from jax.experimental.pallas import tpu as pltpu

N_CLS = 10000
N_CTX = 16
D = 512
SEQ = 77
SUF = 60
B = 1024
G = 16


def _body(idx_ref, *refs):
    pre_refs = refs[:G]
    ctx_ref = refs[G]
    suf_refs = refs[G + 1:2 * G + 1]
    out_ref = refs[2 * G + 1]
    for k in range(G):
        out_ref[k, 0:1, :] = pre_refs[k][0]
        out_ref[k, 1:1 + N_CTX, :] = ctx_ref[...]
        out_ref[k, 1 + N_CTX:, :] = suf_refs[k][0]


def _pre_spec(k):
    return pl.BlockSpec((1, 1, D), lambda g, idx_ref, k=k: (idx_ref[G * g + k], 0, 0))


def _suf_spec(k):
    return pl.BlockSpec((1, SUF, D), lambda g, idx_ref, k=k: (idx_ref[G * g + k], 0, 0))


@jax.jit
def _launch(idx, ctx, token_prefix, token_suffix):
    grid_spec = pltpu.PrefetchScalarGridSpec(
        num_scalar_prefetch=1,
        grid=(B // G,),
        in_specs=[
            *[_pre_spec(k) for k in range(G)],
            pl.BlockSpec((N_CTX, D), lambda g, idx_ref: (0, 0)),
            *[_suf_spec(k) for k in range(G)],
        ],
        out_specs=pl.BlockSpec((G, SEQ, D), lambda g, idx_ref: (g, 0, 0)),
    )
    return pl.pallas_call(
        _body,
        grid_spec=grid_spec,
        out_shape=jax.ShapeDtypeStruct((B, SEQ, D), jnp.float32),
    )(idx, *([token_prefix] * G), ctx, *([token_suffix] * G))


def kernel(idx, ctx, token_prefix, token_suffix):
    return _launch(idx, ctx, token_prefix, token_suffix)
